# baseline (speedup 1.0000x reference)
import functools

import jax
import jax.numpy as jnp
import numpy as np
from jax import lax
from jax.experimental import pallas as pl
from jax.experimental.pallas import tpu as pltpu
from jax.experimental.pallas import tpu_sc as plsc

N = 10000
E = 320000
D_IN = 128
D_H = 64

NC = 2
NS = 16
NW = NC * NS

N_PAD = 10240
ROWS_PS = N_PAD // NS
CHUNK = 128
GRP = 1
NG = 80
E_PAD = NW * NG * GRP * CHUNK
BLK = 2048
GRID = N_PAD // BLK

NB = 8
PF = 4

_mesh = plsc.VectorSubcoreMesh(core_axis_name="c", subcore_axis_name="s")

_NPAD_E = E_PAD - E
_PAD_SRC = np.arange(_NPAD_E, dtype=np.int32) % N
_PAD_DST = (N + np.arange(_NPAD_E, dtype=np.int32) % (N_PAD - N)).astype(np.int32)


@functools.partial(
    pl.kernel,
    out_type=jax.ShapeDtypeStruct((NC * N_PAD,), jnp.float32),
    mesh=_mesh,
    scratch_types=[
        pltpu.VMEM((NG, CHUNK), jnp.int32),
        pltpu.VMEM((CHUNK,), jnp.float32),
        pltpu.VMEM_SHARED((N_PAD,), jnp.float32),
        pltpu.SemaphoreType.DMA,
    ],
)
def _sc_hist(dst_hbm, ones_hbm, zeros_hbm, hist_hbm, dst_v, ones_v, hist_sp,
             ssem):
    c = lax.axis_index("c")
    s = lax.axis_index("s")
    wid = c * NS + s

    pltpu.sync_copy(zeros_hbm.at[pl.ds(s * ROWS_PS, ROWS_PS)],
                    hist_sp.at[pl.ds(s * ROWS_PS, ROWS_PS)])
    pltpu.sync_copy(ones_hbm, ones_v)
    pltpu.sync_copy(dst_hbm.at[wid], dst_v)
    plsc.subcore_barrier()

    for j0 in range(8):
        pltpu.async_copy(ones_v, hist_sp.at[dst_v.at[j0]], ssem, add=True)

    def step(j, carry):
        pltpu.make_async_copy(ones_v, hist_sp.at[dst_v.at[j]], ssem).wait()
        pltpu.async_copy(ones_v, hist_sp.at[dst_v.at[j + 8]], ssem, add=True)
        return carry

    lax.fori_loop(0, NG - 8, step, 0)
    for j0 in range(8):
        pltpu.make_async_copy(ones_v, hist_sp.at[dst_v.at[j0]], ssem).wait()

    plsc.subcore_barrier()
    pltpu.sync_copy(hist_sp.at[pl.ds(s * ROWS_PS, ROWS_PS)],
                    hist_hbm.at[pl.ds(c * N_PAD + s * ROWS_PS, ROWS_PS)])


@functools.partial(
    pl.kernel,
    out_type=jax.ShapeDtypeStruct((NC * N_PAD, D_H), jnp.float32),
    mesh=_mesh,
    scratch_types=[
        pltpu.VMEM((NG, CHUNK), jnp.int32),
        pltpu.VMEM((NG, CHUNK), jnp.int32),
        pltpu.VMEM((NB, CHUNK, D_H), jnp.float32),
        pltpu.VMEM_SHARED((N_PAD, D_H), jnp.float32),
        pltpu.SemaphoreType.DMA,
        pltpu.SemaphoreType.DMA,
    ],
    compiler_params=pltpu.CompilerParams(use_tc_tiling_on_sc=False),
)
def _sc_segsum(y_hbm, src_hbm, dst_hbm, zeros_hbm, z_hbm,
               src_v, dst_v, gbuf, z_sp, gsem, ssem):
    c = lax.axis_index("c")
    s = lax.axis_index("s")
    wid = c * NS + s

    pltpu.sync_copy(zeros_hbm.at[pl.ds(s * ROWS_PS, ROWS_PS)],
                    z_sp.at[pl.ds(s * ROWS_PS, ROWS_PS)])
    pltpu.sync_copy(src_hbm.at[wid], src_v)
    pltpu.sync_copy(dst_hbm.at[wid], dst_v)
    plsc.subcore_barrier()

    for b0 in range(PF):
        pltpu.async_copy(y_hbm.at[src_v.at[b0]], gbuf.at[b0], gsem)

    def step(j, carry):
        for b in range(NB):
            g = NB * j + b
            pltpu.make_async_copy(y_hbm.at[src_v.at[g]], gbuf.at[b],
                                  gsem).wait()
            pltpu.async_copy(gbuf.at[b], z_sp.at[dst_v.at[g]], ssem,
                             add=True)
            @pl.when(g >= NB - PF)
            def _():
                pltpu.make_async_copy(gbuf.at[b], z_sp.at[dst_v.at[g]],
                                      ssem).wait()
            @pl.when(g + PF < NG)
            def _():
                pltpu.async_copy(y_hbm.at[src_v.at[g + PF]],
                                 gbuf.at[(b + PF) % NB], gsem)
        return carry

    lax.fori_loop(0, NG // NB, step, 0)
    for _ in range(NB - PF):
        pltpu.make_async_copy(gbuf.at[0], z_sp.at[dst_v.at[0]], ssem).wait()

    plsc.subcore_barrier()
    pltpu.sync_copy(z_sp.at[pl.ds(s * ROWS_PS, ROWS_PS)],
                    z_hbm.at[pl.ds(c * N_PAD + s * ROWS_PS, ROWS_PS)])


def _dinv_col(hist_ref):
    deg = hist_ref[0:1, :] + hist_ref[1:2, :] + 1.0
    return lax.rsqrt(deg).reshape(BLK, 1)


def _tc_scale_matmul_body(hist_ref, x_ref, w_ref, y_ref):
    y_ref[...] = jnp.dot(x_ref[...], w_ref[...],
                         preferred_element_type=jnp.float32) * _dinv_col(hist_ref)


def _tc_mid_body(hist_ref, z0_ref, z1_ref, y_ref, w_ref, b_ref, y2_ref):
    dinv = _dinv_col(hist_ref)
    zsum = z0_ref[...] + z1_ref[...] + y_ref[...]
    h = jnp.maximum(dinv * zsum + b_ref[...], 0.0)
    y2_ref[...] = jnp.dot(h, w_ref[...],
                          preferred_element_type=jnp.float32) * dinv


def _tc_final_body(hist_ref, z0_ref, z1_ref, y_ref, b_ref, out_ref):
    zsum = z0_ref[...] + z1_ref[...] + y_ref[...]
    out_ref[...] = jnp.maximum(_dinv_col(hist_ref) * zsum + b_ref[...], 0.0)


_hist_spec = pl.BlockSpec((2, BLK), lambda i: (0, i))
_row_spec = pl.BlockSpec((BLK, D_H), lambda i: (i, 0))
_z0_spec = pl.BlockSpec((BLK, D_H), lambda i: (i, 0))
_z1_spec = pl.BlockSpec((BLK, D_H), lambda i: (i + GRID, 0))
_b_spec = pl.BlockSpec((1, D_H), lambda i: (0, 0))

_tc_scale_matmul = pl.pallas_call(
    _tc_scale_matmul_body,
    grid=(GRID,),
    in_specs=[_hist_spec,
              pl.BlockSpec((BLK, D_IN), lambda i: (i, 0)),
              pl.BlockSpec((D_IN, D_H), lambda i: (0, 0))],
    out_specs=_row_spec,
    out_shape=jax.ShapeDtypeStruct((N, D_H), jnp.float32),
)

_tc_mid = pl.pallas_call(
    _tc_mid_body,
    grid=(GRID,),
    in_specs=[_hist_spec, _z0_spec, _z1_spec, _row_spec,
              pl.BlockSpec((D_H, D_H), lambda i: (0, 0)), _b_spec],
    out_specs=_row_spec,
    out_shape=jax.ShapeDtypeStruct((N, D_H), jnp.float32),
)

_tc_final = pl.pallas_call(
    _tc_final_body,
    grid=(GRID,),
    in_specs=[_hist_spec, _z0_spec, _z1_spec, _row_spec, _b_spec],
    out_specs=_row_spec,
    out_shape=jax.ShapeDtypeStruct((N, D_H), jnp.float32),
)


def kernel(x, edge_index, W1, b1, W2, b2):
    src = edge_index[0]
    dst = edge_index[1]

    src_p = jnp.concatenate([src, _PAD_SRC]).reshape(NW, NG, CHUNK)
    dst_p = jnp.concatenate([dst, _PAD_DST]).reshape(NW, NG, CHUNK)

    zeros1 = jnp.zeros((N_PAD,), jnp.float32)
    zeros2 = jnp.zeros((N_PAD, D_H), jnp.float32)
    ones_c = jnp.ones((CHUNK,), jnp.float32)

    hist = _sc_hist(dst_p, ones_c, zeros1).reshape(NC, N_PAD)

    y1 = _tc_scale_matmul(hist, x, W1)
    z1 = _sc_segsum(y1, src_p, dst_p, zeros2)
    y2 = _tc_mid(hist, z1, z1, y1, W2, b1.reshape(1, D_H))
    z2 = _sc_segsum(y2, src_p, dst_p, zeros2)
    return _tc_final(hist, z2, z2, y2, b2.reshape(1, D_H))


# speedup vs baseline: 1.0588x; 1.0588x over previous
import functools

import jax
import jax.numpy as jnp
from jax import lax
from jax.experimental import pallas as pl
from jax.experimental.pallas import tpu as pltpu
from jax.experimental.pallas import tpu_sc as plsc

N = 10000
E = 320000
D_IN = 128
D_H = 64

NC = 2
NS = 16
NW = NC * NS

N_PAD = 10240
ROWS_PS = N_PAD // NS
CHUNK = 128
EPW = E // NW
NG = EPW // CHUNK
TAIL = EPW - NG * CHUNK
BLK = 2048
GRID = N_PAD // BLK

NB = 6
PF = 4

_mesh = plsc.VectorSubcoreMesh(core_axis_name="c", subcore_axis_name="s")


@functools.partial(
    pl.kernel,
    out_type=jax.ShapeDtypeStruct((NC * N_PAD,), jnp.float32),
    mesh=_mesh,
    scratch_types=[
        pltpu.VMEM((EPW,), jnp.int32),
        pltpu.VMEM((CHUNK,), jnp.float32),
        pltpu.VMEM_SHARED((N_PAD,), jnp.float32),
        pltpu.SemaphoreType.DMA,
    ],
)
def _sc_hist(ei_hbm, ones_hbm, zeros_hbm, hist_hbm, dst_v, ones_v, hist_sp,
             ssem):
    c = lax.axis_index("c")
    s = lax.axis_index("s")
    wid = c * NS + s

    pltpu.sync_copy(zeros_hbm.at[pl.ds(s * ROWS_PS, ROWS_PS)],
                    hist_sp.at[pl.ds(s * ROWS_PS, ROWS_PS)])
    pltpu.sync_copy(ones_hbm, ones_v)
    pltpu.sync_copy(ei_hbm.at[pl.ds(E + wid * EPW, EPW)], dst_v)
    plsc.subcore_barrier()

    for j0 in range(8):
        pltpu.async_copy(ones_v, hist_sp.at[dst_v.at[pl.ds(j0 * CHUNK, CHUNK)]],
                         ssem, add=True)

    def step(j, carry):
        pltpu.make_async_copy(
            ones_v, hist_sp.at[dst_v.at[pl.ds(0, CHUNK)]], ssem).wait()
        pltpu.async_copy(
            ones_v, hist_sp.at[dst_v.at[pl.ds((j + 8) * CHUNK, CHUNK)]],
            ssem, add=True)
        return carry

    lax.fori_loop(0, NG - 8, step, 0)
    for _ in range(8):
        pltpu.make_async_copy(
            ones_v, hist_sp.at[dst_v.at[pl.ds(0, CHUNK)]], ssem).wait()
    pltpu.sync_copy(ones_v.at[pl.ds(0, TAIL)],
                    hist_sp.at[dst_v.at[pl.ds(NG * CHUNK, TAIL)]], add=True)

    plsc.subcore_barrier()
    pltpu.sync_copy(hist_sp.at[pl.ds(s * ROWS_PS, ROWS_PS)],
                    hist_hbm.at[pl.ds(c * N_PAD + s * ROWS_PS, ROWS_PS)])


@functools.partial(
    pl.kernel,
    out_type=jax.ShapeDtypeStruct((NC * N_PAD, D_H), jnp.float32),
    mesh=_mesh,
    scratch_types=[
        pltpu.VMEM((EPW,), jnp.int32),
        pltpu.VMEM((EPW,), jnp.int32),
        pltpu.VMEM((NB, CHUNK, D_H), jnp.float32),
        pltpu.VMEM_SHARED((N_PAD, D_H), jnp.float32),
        pltpu.SemaphoreType.DMA,
        pltpu.SemaphoreType.DMA,
    ],
    compiler_params=pltpu.CompilerParams(use_tc_tiling_on_sc=False),
)
def _sc_segsum(y_hbm, ei_hbm, zeros_hbm, z_hbm,
               src_v, dst_v, gbuf, z_sp, gsem, ssem):
    c = lax.axis_index("c")
    s = lax.axis_index("s")
    wid = c * NS + s

    pltpu.sync_copy(zeros_hbm.at[pl.ds(s * ROWS_PS, ROWS_PS)],
                    z_sp.at[pl.ds(s * ROWS_PS, ROWS_PS)])
    pltpu.sync_copy(ei_hbm.at[pl.ds(wid * EPW, EPW)], src_v)
    pltpu.sync_copy(ei_hbm.at[pl.ds(E + wid * EPW, EPW)], dst_v)
    plsc.subcore_barrier()

    def src_at(g):
        return src_v.at[pl.ds(g * CHUNK, CHUNK)]

    def dst_at(g):
        return dst_v.at[pl.ds(g * CHUNK, CHUNK)]

    for b0 in range(PF):
        pltpu.async_copy(y_hbm.at[src_at(b0)], gbuf.at[b0], gsem)

    def step(j, carry):
        for b in range(NB):
            g = NB * j + b
            pltpu.make_async_copy(y_hbm.at[src_at(g)], gbuf.at[b],
                                  gsem).wait()
            pltpu.async_copy(gbuf.at[b], z_sp.at[dst_at(g)], ssem,
                             add=True)
            @pl.when(g >= NB - PF)
            def _():
                pltpu.make_async_copy(gbuf.at[b], z_sp.at[dst_at(g)],
                                      ssem).wait()
            @pl.when(g + PF < NG)
            def _():
                pltpu.async_copy(y_hbm.at[src_at(g + PF)],
                                 gbuf.at[(b + PF) % NB], gsem)
        return carry

    lax.fori_loop(0, NG // NB, step, 0)
    for _ in range(NB - PF):
        pltpu.make_async_copy(gbuf.at[0], z_sp.at[dst_at(0)], ssem).wait()
    pltpu.async_copy(y_hbm.at[src_v.at[pl.ds(NG * CHUNK, TAIL)]],
                     gbuf.at[0, pl.ds(0, TAIL)], gsem)
    pltpu.make_async_copy(y_hbm.at[src_v.at[pl.ds(NG * CHUNK, TAIL)]],
                          gbuf.at[0, pl.ds(0, TAIL)], gsem).wait()
    pltpu.sync_copy(gbuf.at[0, pl.ds(0, TAIL)],
                    z_sp.at[dst_v.at[pl.ds(NG * CHUNK, TAIL)]], add=True)

    plsc.subcore_barrier()
    pltpu.sync_copy(z_sp.at[pl.ds(s * ROWS_PS, ROWS_PS)],
                    z_hbm.at[pl.ds(c * N_PAD + s * ROWS_PS, ROWS_PS)])


def _dinv_col(hist_ref):
    deg = hist_ref[0:1, :] + hist_ref[1:2, :] + 1.0
    return lax.rsqrt(deg).reshape(BLK, 1)


def _tc_scale_matmul_body(hist_ref, x_ref, w_ref, y_ref):
    y_ref[...] = jnp.dot(x_ref[...], w_ref[...],
                         preferred_element_type=jnp.float32) * _dinv_col(hist_ref)


def _tc_mid_body(hist_ref, z0_ref, z1_ref, y_ref, w_ref, b_ref, y2_ref):
    dinv = _dinv_col(hist_ref)
    zsum = z0_ref[...] + z1_ref[...] + y_ref[...]
    h = jnp.maximum(dinv * zsum + b_ref[...], 0.0)
    y2_ref[...] = jnp.dot(h, w_ref[...],
                          preferred_element_type=jnp.float32) * dinv


def _tc_final_body(hist_ref, z0_ref, z1_ref, y_ref, b_ref, out_ref):
    zsum = z0_ref[...] + z1_ref[...] + y_ref[...]
    out_ref[...] = jnp.maximum(_dinv_col(hist_ref) * zsum + b_ref[...], 0.0)


_hist_spec = pl.BlockSpec((2, BLK), lambda i: (0, i))
_row_spec = pl.BlockSpec((BLK, D_H), lambda i: (i, 0))
_z0_spec = pl.BlockSpec((BLK, D_H), lambda i: (i, 0))
_z1_spec = pl.BlockSpec((BLK, D_H), lambda i: (i + GRID, 0))
_b_spec = pl.BlockSpec((1, D_H), lambda i: (0, 0))

_tc_scale_matmul = pl.pallas_call(
    _tc_scale_matmul_body,
    grid=(GRID,),
    in_specs=[_hist_spec,
              pl.BlockSpec((BLK, D_IN), lambda i: (i, 0)),
              pl.BlockSpec((D_IN, D_H), lambda i: (0, 0))],
    out_specs=_row_spec,
    out_shape=jax.ShapeDtypeStruct((N, D_H), jnp.float32),
)

_tc_mid = pl.pallas_call(
    _tc_mid_body,
    grid=(GRID,),
    in_specs=[_hist_spec, _z0_spec, _z1_spec, _row_spec,
              pl.BlockSpec((D_H, D_H), lambda i: (0, 0)), _b_spec],
    out_specs=_row_spec,
    out_shape=jax.ShapeDtypeStruct((N, D_H), jnp.float32),
)

_tc_final = pl.pallas_call(
    _tc_final_body,
    grid=(GRID,),
    in_specs=[_hist_spec, _z0_spec, _z1_spec, _row_spec, _b_spec],
    out_specs=_row_spec,
    out_shape=jax.ShapeDtypeStruct((N, D_H), jnp.float32),
)


def kernel(x, edge_index, W1, b1, W2, b2):
    ei_flat = edge_index.reshape(2 * E)
    zeros1 = jnp.zeros((N_PAD,), jnp.float32)
    zeros2 = jnp.zeros((N_PAD, D_H), jnp.float32)
    ones_c = jnp.ones((CHUNK,), jnp.float32)

    hist = _sc_hist(ei_flat, ones_c, zeros1).reshape(NC, N_PAD)

    y1 = _tc_scale_matmul(hist, x, W1)
    z1 = _sc_segsum(y1, ei_flat, zeros2)
    y2 = _tc_mid(hist, z1, z1, y1, W2, b1.reshape(1, D_H))
    z2 = _sc_segsum(y2, ei_flat, zeros2)
    return _tc_final(hist, z2, z2, y2, b2.reshape(1, D_H))
